# Initial kernel scaffold; baseline (speedup 1.0000x reference)
import jax
import jax.numpy as jnp
from jax.experimental import pallas as pl


def kernel(x, edge_index, pseudo, W1, R1, b1, W2, R2, b2, W3, R3, b3, W4, R4, b4, W5, R5, b5, W6, R6, b6, fc1_w, fc1_b, fc2_w, fc2_b):
    raise NotImplementedError("write your pallas kernel here")


# speedup vs baseline: 5.8735x; 5.8735x over previous
import functools

import jax
import jax.numpy as jnp
from jax import lax
from jax.experimental import pallas as pl
from jax.experimental.pallas import tpu as pltpu
from jax.experimental.pallas import tpu_sc as plsc

_K = 5
_DIM = 3
_KD = _K ** _DIM

_NC = 2
_NS = 16
_NT = _NC * _NS
_G = 16

_f32 = jnp.float32
_i32 = jnp.int32


def _precompute_body(pt_ref, ei_ref, b8_ref, rows8_ref):
    v = pt_ref[...] * float(_K - 1)
    lo_f = jnp.floor(v)
    frac = v - lo_f
    lo = jnp.clip(lo_f.astype(_i32), 0, _K - 2)
    src = ei_ref[0:1, :]
    bs, rs = [], []
    for c in range(2 ** _DIM):
        B = jnp.ones_like(src, dtype=_f32)
        idx = jnp.zeros_like(src)
        for d in range(_DIM):
            bd = (c >> d) & 1
            fd = frac[d:d + 1, :]
            B = B * (fd if bd else (1.0 - fd))
            idx = idx + (lo[d:d + 1, :] + bd) * (_K ** d)
        bs.append(B)
        rs.append(src * _KD + idx)
    b8_ref[...] = jnp.concatenate(bs, axis=0)
    rows8_ref[...] = jnp.concatenate(rs, axis=0)


def _precompute(pseudo_t, edge_index):
    E = pseudo_t.shape[1]
    EB = 6400
    n_blk = E // EB
    return pl.pallas_call(
        _precompute_body,
        grid=(n_blk,),
        in_specs=[
            pl.BlockSpec((3, EB), lambda i: (0, i)),
            pl.BlockSpec((2, EB), lambda i: (0, i)),
        ],
        out_specs=[
            pl.BlockSpec((8, EB), lambda i: (0, i)),
            pl.BlockSpec((8, EB), lambda i: (0, i)),
        ],
        out_shape=[
            jax.ShapeDtypeStruct((8, E), _f32),
            jax.ShapeDtypeStruct((8, E), _i32),
        ],
    )(pseudo_t, edge_index)


def _mm_body(a_ref, w_ref, o_ref):
    o_ref[...] = jnp.dot(a_ref[...], w_ref[...], preferred_element_type=_f32)


def _xw_matmul(h, wf):
    n, ci = h.shape
    co = wf.shape[1]
    RB = 400
    return pl.pallas_call(
        _mm_body,
        grid=(n // RB,),
        in_specs=[
            pl.BlockSpec((RB, ci), lambda i: (i, 0)),
            pl.BlockSpec((ci, co), lambda i: (0, 0)),
        ],
        out_specs=pl.BlockSpec((RB, co), lambda i: (i, 0)),
        out_shape=jax.ShapeDtypeStruct((n, co), _f32),
    )(h, wf)


def _bcast_body(x_ref, w_ref, o_ref):
    o_ref[...] = x_ref[...] * w_ref[...]


def _xw_bcast(x, wf):
    n = x.shape[0]
    co = wf.shape[1]
    RB = 1000
    return pl.pallas_call(
        _bcast_body,
        grid=(n // RB,),
        in_specs=[
            pl.BlockSpec((RB, 1), lambda i: (i, 0)),
            pl.BlockSpec((1, co), lambda i: (0, 0)),
        ],
        out_specs=pl.BlockSpec((RB, co), lambda i: (i, 0)),
        out_shape=jax.ShapeDtypeStruct((n, co), _f32),
    )(x, wf)


def _post_body(m0_ref, m1_ref, d0_ref, d1_ref, h_ref, r_ref, b_ref, o_ref):
    deg = jnp.maximum(d0_ref[:, 0:1] + d1_ref[:, 0:1], 1.0)
    msg = (m0_ref[...] + m1_ref[...]) / deg
    y = msg + jnp.dot(h_ref[...], r_ref[...],
                      preferred_element_type=_f32) + b_ref[...]
    o_ref[...] = jnp.where(y > 0, y, jnp.exp(y) - 1.0)


def _post(m0, m1, d0, d1, h, r, b):
    n, co = m0.shape
    ci = h.shape[1]
    RB = 400
    return pl.pallas_call(
        _post_body,
        grid=(n // RB,),
        in_specs=[
            pl.BlockSpec((RB, co), lambda i: (i, 0)),
            pl.BlockSpec((RB, co), lambda i: (i, 0)),
            pl.BlockSpec((RB, 16), lambda i: (i, 0)),
            pl.BlockSpec((RB, 16), lambda i: (i, 0)),
            pl.BlockSpec((RB, ci), lambda i: (i, 0)),
            pl.BlockSpec((ci, co), lambda i: (0, 0)),
            pl.BlockSpec((1, co), lambda i: (0, 0)),
        ],
        out_specs=pl.BlockSpec((RB, co), lambda i: (i, 0)),
        out_shape=jax.ShapeDtypeStruct((n, co), _f32),
    )(m0, m1, d0, d1, h, r, b)


def _head_body(h_ref, w1_ref, b1_ref, w2_ref, b2_ref, o_ref):
    a = jnp.dot(h_ref[...], w1_ref[...], preferred_element_type=_f32) \
        + b1_ref[...]
    a = jnp.where(a > 0, a, jnp.exp(a) - 1.0)
    z = jnp.dot(a, w2_ref[...], preferred_element_type=_f32) + b2_ref[...]
    m = jnp.max(z, axis=1, keepdims=True)
    lse = m + jnp.log(jnp.sum(jnp.exp(z - m), axis=1, keepdims=True))
    o_ref[...] = z - lse


def _head(h, w1, b1, w2, b2):
    n, ci = h.shape
    cm = w1.shape[1]
    co = w2.shape[1]
    RB = 400
    return pl.pallas_call(
        _head_body,
        grid=(n // RB,),
        in_specs=[
            pl.BlockSpec((RB, ci), lambda i: (i, 0)),
            pl.BlockSpec((ci, cm), lambda i: (0, 0)),
            pl.BlockSpec((1, cm), lambda i: (0, 0)),
            pl.BlockSpec((cm, co), lambda i: (0, 0)),
            pl.BlockSpec((1, co), lambda i: (0, 0)),
        ],
        out_specs=pl.BlockSpec((RB, co), lambda i: (i, 0)),
        out_shape=jax.ShapeDtypeStruct((n, co), _f32),
    )(h, w1, b1, w2, b2)


def _sc_mesh():
    return plsc.VectorSubcoreMesh(core_axis_name="c", subcore_axis_name="s",
                                  num_cores=_NC, num_subcores=_NS)


_S = 16


@functools.lru_cache(maxsize=None)
def _make_msg_kernel(n_pad, ch, d):
    rows_per_sub = n_pad // _NS

    @functools.partial(
        pl.kernel,
        mesh=_sc_mesh(),
        compiler_params=pltpu.CompilerParams(use_tc_tiling_on_sc=False),
        out_type=jax.ShapeDtypeStruct((_NC, n_pad, d), _f32),
        scratch_types=[
            pltpu.VMEM((_S, 8 * _G), _i32),
            pltpu.VMEM((_S, 8 * _G), _f32),
            pltpu.VMEM((_S, _G), _i32),
            pltpu.VMEM((8 * _G, d), _f32),
            pltpu.VMEM((_G, d), _f32),
            pltpu.VMEM_SHARED((n_pad, d), _f32),
            pltpu.SemaphoreType.DMA,
        ],
    )
    def kern(table, rows_t, b_t, dst_t, out, idx_v, b_v, dst_v, g_v, m_v,
             msg_sh, sem):
        cid = lax.axis_index("c")
        sid = lax.axis_index("s")
        wid = sid * _NC + cid

        zero = jnp.zeros((16,), _f32)
        for j in range(_G):
            for q in range(d // 16):
                m_v[j, pl.ds(16 * q, 16)] = zero

        base = sid * rows_per_sub

        def zero_body(i, carry):
            pltpu.sync_copy(m_v, msg_sh.at[pl.ds(base + i * _G, _G)])
            return carry

        lax.fori_loop(0, rows_per_sub // _G, zero_body, 0)
        plsc.subcore_barrier()

        def group_body(gi, carry):
            pltpu.sync_copy(rows_t.at[wid, pl.ds(gi * _S, _S)], idx_v)
            pltpu.sync_copy(b_t.at[wid, pl.ds(gi * _S, _S)], b_v)
            pltpu.sync_copy(dst_t.at[wid, pl.ds(gi * _S, _S)], dst_v)

            def body(s_i, carry2):
                pltpu.async_copy(table.at[idx_v.at[s_i]], g_v, sem).wait()
                for jj in range(_G // 2):
                    bv = b_v[s_i, pl.ds(16 * jj, 16)]
                    for e in range(2):
                        j = 2 * jj + e
                        ws = [bv[8 * e + c] for c in range(8)]
                        for q in range(d // 16):
                            acc = g_v[8 * j, pl.ds(16 * q, 16)] * ws[0]
                            for c in range(1, 8):
                                acc = acc + g_v[8 * j + c,
                                                pl.ds(16 * q, 16)] * ws[c]
                            m_v[j, pl.ds(16 * q, 16)] = acc
                pltpu.sync_copy(m_v, msg_sh.at[dst_v.at[s_i]], add=True)
                return carry2

            lax.fori_loop(0, _S, body, 0)
            return carry

        lax.fori_loop(0, ch // _S, group_body, 0)
        plsc.subcore_barrier()
        pltpu.sync_copy(msg_sh.at[pl.ds(base, rows_per_sub)],
                        out.at[cid, pl.ds(base, rows_per_sub)])

    return kern


@functools.lru_cache(maxsize=None)
def _make_deg_kernel(n_pad, ch):
    rows_per_sub = n_pad // _NS

    @functools.partial(
        pl.kernel,
        mesh=_sc_mesh(),
        compiler_params=pltpu.CompilerParams(use_tc_tiling_on_sc=False),
        out_type=jax.ShapeDtypeStruct((_NC, n_pad, 16), _f32),
        scratch_types=[
            pltpu.VMEM((_S, _G), _i32),
            pltpu.VMEM((_G, 16), _f32),
            pltpu.VMEM_SHARED((n_pad, 16), _f32),
        ],
    )
    def kern(dst_t, out, dst_v, ones_v, deg_sh):
        cid = lax.axis_index("c")
        sid = lax.axis_index("s")
        wid = sid * _NC + cid

        zero = jnp.zeros((16,), _f32)
        for j in range(_G):
            ones_v[j, :] = zero

        base = sid * rows_per_sub

        def zero_body(i, carry):
            pltpu.sync_copy(ones_v, deg_sh.at[pl.ds(base + i * _G, _G)])
            return carry

        lax.fori_loop(0, rows_per_sub // _G, zero_body, 0)

        lane = lax.iota(_i32, 16)
        onerow = jnp.where(lane == 0, 1.0, 0.0).astype(_f32)
        for j in range(_G):
            ones_v[j, :] = onerow
        plsc.subcore_barrier()

        def group_body(gi, carry):
            pltpu.sync_copy(dst_t.at[wid, pl.ds(gi * _S, _S)], dst_v)

            def body(s_i, carry2):
                pltpu.sync_copy(ones_v, deg_sh.at[dst_v.at[s_i]], add=True)
                return carry2

            lax.fori_loop(0, _S, body, 0)
            return carry

        lax.fori_loop(0, ch // _S, group_body, 0)
        plsc.subcore_barrier()
        pltpu.sync_copy(deg_sh.at[pl.ds(base, rows_per_sub)],
                        out.at[cid, pl.ds(base, rows_per_sub)])

    return kern


def kernel(x, edge_index, pseudo, W1, R1, b1, W2, R2, b2, W3, R3, b3,
           W4, R4, b4, W5, R5, b5, W6, R6, b6, fc1_w, fc1_b, fc2_w, fc2_b):
    N = x.shape[0]
    E = edge_index.shape[1]
    CH = -(-E // (_NT * _G * _S)) * _S
    Ep = _NT * _G * CH
    n_pad = -(-N // (_NS * _G)) * (_NS * _G)

    b8, rows8 = _precompute(pseudo.T, edge_index)
    padE = Ep - E
    b_t = jnp.pad(b8, ((0, 0), (0, padE))).T.reshape(_NT, CH, 8 * _G)
    rows_t = jnp.pad(rows8, ((0, 0), (0, padE))).T.reshape(_NT, CH, 8 * _G)
    dst_t = jnp.pad(edge_index[1], (0, padE),
                    constant_values=N).reshape(_NT, CH, _G)

    degp = _make_deg_kernel(n_pad, CH)(dst_t)
    d0 = degp[0, :N]
    d1 = degp[1, :N]

    layers = [(W1, R1, b1), (W2, R2, b2), (W3, R3, b3),
              (W4, R4, b4), (W5, R5, b5), (W6, R6, b6)]
    h = x
    for li, (W, R, b) in enumerate(layers):
        ci, co = W.shape[1], W.shape[2]
        wf = W.transpose(1, 0, 2).reshape(ci, _KD * co)
        if ci == 1:
            xw = _xw_bcast(h, wf)
        else:
            xw = _xw_matmul(h, wf)
        table = xw.reshape(N * _KD, co)
        msgp = _make_msg_kernel(n_pad, CH, co)(table, rows_t, b_t, dst_t)
        h = _post(msgp[0, :N], msgp[1, :N], d0, d1, h, R, b.reshape(1, co))

    return _head(h, fc1_w, fc1_b.reshape(1, -1), fc2_w, fc2_b.reshape(1, -1))


# speedup vs baseline: 7.5805x; 1.2906x over previous
import functools

import jax
import jax.numpy as jnp
from jax import lax
from jax.experimental import pallas as pl
from jax.experimental.pallas import tpu as pltpu
from jax.experimental.pallas import tpu_sc as plsc

_K = 5
_DIM = 3
_KD = _K ** _DIM

_NC = 2
_NS = 16
_NT = _NC * _NS
_G = 16

_f32 = jnp.float32
_i32 = jnp.int32


def _precompute_body(pt_ref, ei_ref, b8_ref, rows8_ref):
    v = pt_ref[...] * float(_K - 1)
    lo_f = jnp.floor(v)
    frac = v - lo_f
    lo = jnp.clip(lo_f.astype(_i32), 0, _K - 2)
    src = ei_ref[0:1, :]
    bs, rs = [], []
    for c in range(2 ** _DIM):
        B = jnp.ones_like(src, dtype=_f32)
        idx = jnp.zeros_like(src)
        for d in range(_DIM):
            bd = (c >> d) & 1
            fd = frac[d:d + 1, :]
            B = B * (fd if bd else (1.0 - fd))
            idx = idx + (lo[d:d + 1, :] + bd) * (_K ** d)
        bs.append(B)
        rs.append(src * _KD + idx)
    b8_ref[...] = jnp.concatenate(bs, axis=0)
    rows8_ref[...] = jnp.concatenate(rs, axis=0)


def _precompute(pseudo_t, edge_index):
    E = pseudo_t.shape[1]
    EB = 6400
    n_blk = E // EB
    return pl.pallas_call(
        _precompute_body,
        grid=(n_blk,),
        in_specs=[
            pl.BlockSpec((3, EB), lambda i: (0, i)),
            pl.BlockSpec((2, EB), lambda i: (0, i)),
        ],
        out_specs=[
            pl.BlockSpec((8, EB), lambda i: (0, i)),
            pl.BlockSpec((8, EB), lambda i: (0, i)),
        ],
        out_shape=[
            jax.ShapeDtypeStruct((8, E), _f32),
            jax.ShapeDtypeStruct((8, E), _i32),
        ],
    )(pseudo_t, edge_index)


def _mm_body(a_ref, w_ref, o_ref):
    o_ref[...] = jnp.dot(a_ref[...], w_ref[...], preferred_element_type=_f32)


def _xw_matmul(h, wf):
    n, ci = h.shape
    co = wf.shape[1]
    RB = 400
    return pl.pallas_call(
        _mm_body,
        grid=(n // RB,),
        in_specs=[
            pl.BlockSpec((RB, ci), lambda i: (i, 0)),
            pl.BlockSpec((ci, co), lambda i: (0, 0)),
        ],
        out_specs=pl.BlockSpec((RB, co), lambda i: (i, 0)),
        out_shape=jax.ShapeDtypeStruct((n, co), _f32),
    )(h, wf)


def _bcast_body(x_ref, w_ref, o_ref):
    o_ref[...] = x_ref[...] * w_ref[...]


def _xw_bcast(x, wf):
    n = x.shape[0]
    co = wf.shape[1]
    RB = 1000
    return pl.pallas_call(
        _bcast_body,
        grid=(n // RB,),
        in_specs=[
            pl.BlockSpec((RB, 1), lambda i: (i, 0)),
            pl.BlockSpec((1, co), lambda i: (0, 0)),
        ],
        out_specs=pl.BlockSpec((RB, co), lambda i: (i, 0)),
        out_shape=jax.ShapeDtypeStruct((n, co), _f32),
    )(x, wf)


def _post_body(m0_ref, m1_ref, d0_ref, d1_ref, h_ref, r_ref, b_ref, o_ref):
    deg = jnp.maximum(d0_ref[:, 0:1] + d1_ref[:, 0:1], 1.0)
    msg = (m0_ref[...] + m1_ref[...]) / deg
    y = msg + jnp.dot(h_ref[...], r_ref[...],
                      preferred_element_type=_f32) + b_ref[...]
    o_ref[...] = jnp.where(y > 0, y, jnp.exp(y) - 1.0)


def _post(m0, m1, d0, d1, h, r, b):
    n, co = m0.shape
    ci = h.shape[1]
    RB = 400
    return pl.pallas_call(
        _post_body,
        grid=(n // RB,),
        in_specs=[
            pl.BlockSpec((RB, co), lambda i: (i, 0)),
            pl.BlockSpec((RB, co), lambda i: (i, 0)),
            pl.BlockSpec((RB, 16), lambda i: (i, 0)),
            pl.BlockSpec((RB, 16), lambda i: (i, 0)),
            pl.BlockSpec((RB, ci), lambda i: (i, 0)),
            pl.BlockSpec((ci, co), lambda i: (0, 0)),
            pl.BlockSpec((1, co), lambda i: (0, 0)),
        ],
        out_specs=pl.BlockSpec((RB, co), lambda i: (i, 0)),
        out_shape=jax.ShapeDtypeStruct((n, co), _f32),
    )(m0, m1, d0, d1, h, r, b)


def _head_body(h_ref, w1_ref, b1_ref, w2_ref, b2_ref, o_ref):
    a = jnp.dot(h_ref[...], w1_ref[...], preferred_element_type=_f32) \
        + b1_ref[...]
    a = jnp.where(a > 0, a, jnp.exp(a) - 1.0)
    z = jnp.dot(a, w2_ref[...], preferred_element_type=_f32) + b2_ref[...]
    m = jnp.max(z, axis=1, keepdims=True)
    lse = m + jnp.log(jnp.sum(jnp.exp(z - m), axis=1, keepdims=True))
    o_ref[...] = z - lse


def _head(h, w1, b1, w2, b2):
    n, ci = h.shape
    cm = w1.shape[1]
    co = w2.shape[1]
    RB = 400
    return pl.pallas_call(
        _head_body,
        grid=(n // RB,),
        in_specs=[
            pl.BlockSpec((RB, ci), lambda i: (i, 0)),
            pl.BlockSpec((ci, cm), lambda i: (0, 0)),
            pl.BlockSpec((1, cm), lambda i: (0, 0)),
            pl.BlockSpec((cm, co), lambda i: (0, 0)),
            pl.BlockSpec((1, co), lambda i: (0, 0)),
        ],
        out_specs=pl.BlockSpec((RB, co), lambda i: (i, 0)),
        out_shape=jax.ShapeDtypeStruct((n, co), _f32),
    )(h, w1, b1, w2, b2)


def _sc_mesh():
    return plsc.VectorSubcoreMesh(core_axis_name="c", subcore_axis_name="s",
                                  num_cores=_NC, num_subcores=_NS)


_S = 16


_NB = 4


@functools.lru_cache(maxsize=None)
def _make_msg_kernel(n_pad, ch, d):
    rows_per_sub = n_pad // _NS

    @functools.partial(
        pl.kernel,
        mesh=_sc_mesh(),
        compiler_params=pltpu.CompilerParams(use_tc_tiling_on_sc=False),
        out_type=jax.ShapeDtypeStruct((_NC, n_pad, d), _f32),
        scratch_types=[
            [pltpu.VMEM((_S, 8 * _G), _i32)] * 2,
            [pltpu.VMEM((_S, 8 * _G), _f32)] * 2,
            [pltpu.VMEM((_S, _G), _i32)] * 2,
            [pltpu.VMEM((8 * _G, d), _f32)] * _NB,
            [pltpu.VMEM((_G, d), _f32)] * _NB,
            pltpu.VMEM_SHARED((n_pad, d), _f32),
            [pltpu.SemaphoreType.DMA] * 2,
            [pltpu.SemaphoreType.DMA] * _NB,
            [pltpu.SemaphoreType.DMA] * _NB,
        ],
    )
    def kern(table, rows_t, b_t, dst_t, out, idx_v, b_v, dst_v, g_v, m_v,
             msg_sh, sm, sg, ss):
        cid = lax.axis_index("c")
        sid = lax.axis_index("s")
        wid = sid * _NC + cid

        zero = jnp.zeros((16,), _f32)
        for j in range(_G):
            for q in range(d // 16):
                m_v[0][j, pl.ds(16 * q, 16)] = zero

        base = sid * rows_per_sub

        def zero_body(i, carry):
            pltpu.sync_copy(m_v[0], msg_sh.at[pl.ds(base + i * _G, _G)])
            return carry

        lax.fori_loop(0, rows_per_sub // _G, zero_body, 0)
        plsc.subcore_barrier()

        def meta_issue(gi, p):
            pltpu.async_copy(rows_t.at[wid, pl.ds(gi * _S, _S)],
                             idx_v[p], sm[p])
            pltpu.async_copy(b_t.at[wid, pl.ds(gi * _S, _S)], b_v[p], sm[p])
            pltpu.async_copy(dst_t.at[wid, pl.ds(gi * _S, _S)],
                             dst_v[p], sm[p])

        def meta_drain(p):
            pltpu.make_async_copy(rows_t.at[wid, pl.ds(0, _S)],
                                  idx_v[p], sm[p]).wait()
            pltpu.make_async_copy(b_t.at[wid, pl.ds(0, _S)],
                                  b_v[p], sm[p]).wait()
            pltpu.make_async_copy(dst_t.at[wid, pl.ds(0, _S)],
                                  dst_v[p], sm[p]).wait()

        def gather_issue(idxb, s_i, b_i):
            pltpu.async_copy(table.at[idxb.at[s_i]], g_v[b_i], sg[b_i])

        def gather_wait(idxb, b_i):
            pltpu.make_async_copy(table.at[idxb.at[0]], g_v[b_i],
                                  sg[b_i]).wait()

        def scatter_issue(dstb, s_i, b_i):
            pltpu.async_copy(m_v[b_i], msg_sh.at[dstb.at[s_i]], ss[b_i],
                             add=True)

        def scatter_wait(dstb, b_i):
            pltpu.make_async_copy(m_v[b_i], msg_sh.at[dstb.at[0]],
                                  ss[b_i]).wait()

        def compute_chunk(bb, s_i, b_i):
            gb = g_v[b_i]
            mb = m_v[b_i]
            for jj in range(_G // 2):
                bv = bb[s_i, pl.ds(16 * jj, 16)]
                for e in range(2):
                    j = 2 * jj + e
                    ws = [bv[8 * e + c] for c in range(8)]
                    for q in range(d // 16):
                        acc = gb[8 * j, pl.ds(16 * q, 16)] * ws[0]
                        for c in range(1, 8):
                            acc = acc + gb[8 * j + c,
                                           pl.ds(16 * q, 16)] * ws[c]
                        mb[j, pl.ds(16 * q, 16)] = acc

        def process_group(idxb, bb, dstb):
            for i in range(_NB - 1):
                gather_issue(idxb, i, i)

            def body(t, carry):
                for i in range(_NB):
                    c_i = t * _NB + i
                    gather_wait(idxb, i)

                    @pl.when(t > 0)
                    def _():
                        scatter_wait(dstb, i)

                    compute_chunk(bb, c_i, i)
                    scatter_issue(dstb, c_i, i)
                    nxt = c_i + _NB - 1

                    @pl.when(nxt < _S)
                    def _():
                        gather_issue(idxb, nxt, (i + _NB - 1) % _NB)
                return carry

            lax.fori_loop(0, _S // _NB, body, 0)
            for i in range(_NB):
                scatter_wait(dstb, i)

        ng = ch // _S
        meta_issue(0, 0)

        def outer(k, carry):
            meta_issue(2 * k + 1, 1)
            meta_drain(0)
            process_group(idx_v[0], b_v[0], dst_v[0])
            meta_issue(2 * k + 2, 0)
            meta_drain(1)
            process_group(idx_v[1], b_v[1], dst_v[1])
            return carry

        lax.fori_loop(0, ng // 2, outer, 0)
        meta_drain(0)
        plsc.subcore_barrier()
        pltpu.sync_copy(msg_sh.at[pl.ds(base, rows_per_sub)],
                        out.at[cid, pl.ds(base, rows_per_sub)])

    return kern


@functools.lru_cache(maxsize=None)
def _make_deg_kernel(n_pad, ch):
    rows_per_sub = n_pad // _NS

    @functools.partial(
        pl.kernel,
        mesh=_sc_mesh(),
        compiler_params=pltpu.CompilerParams(use_tc_tiling_on_sc=False),
        out_type=jax.ShapeDtypeStruct((_NC, n_pad, 16), _f32),
        scratch_types=[
            pltpu.VMEM((_S, _G), _i32),
            pltpu.VMEM((_G, 16), _f32),
            pltpu.VMEM_SHARED((n_pad, 16), _f32),
        ],
    )
    def kern(dst_t, out, dst_v, ones_v, deg_sh):
        cid = lax.axis_index("c")
        sid = lax.axis_index("s")
        wid = sid * _NC + cid

        zero = jnp.zeros((16,), _f32)
        for j in range(_G):
            ones_v[j, :] = zero

        base = sid * rows_per_sub

        def zero_body(i, carry):
            pltpu.sync_copy(ones_v, deg_sh.at[pl.ds(base + i * _G, _G)])
            return carry

        lax.fori_loop(0, rows_per_sub // _G, zero_body, 0)

        lane = lax.iota(_i32, 16)
        onerow = jnp.where(lane == 0, 1.0, 0.0).astype(_f32)
        for j in range(_G):
            ones_v[j, :] = onerow
        plsc.subcore_barrier()

        def group_body(gi, carry):
            pltpu.sync_copy(dst_t.at[wid, pl.ds(gi * _S, _S)], dst_v)

            def body(s_i, carry2):
                pltpu.sync_copy(ones_v, deg_sh.at[dst_v.at[s_i]], add=True)
                return carry2

            lax.fori_loop(0, _S, body, 0)
            return carry

        lax.fori_loop(0, ch // _S, group_body, 0)
        plsc.subcore_barrier()
        pltpu.sync_copy(deg_sh.at[pl.ds(base, rows_per_sub)],
                        out.at[cid, pl.ds(base, rows_per_sub)])

    return kern


def kernel(x, edge_index, pseudo, W1, R1, b1, W2, R2, b2, W3, R3, b3,
           W4, R4, b4, W5, R5, b5, W6, R6, b6, fc1_w, fc1_b, fc2_w, fc2_b):
    N = x.shape[0]
    E = edge_index.shape[1]
    CH = -(-E // (_NT * _G * 2 * _S)) * (2 * _S)
    Ep = _NT * _G * CH
    n_pad = -(-N // (_NS * _G)) * (_NS * _G)

    b8, rows8 = _precompute(pseudo.T, edge_index)
    padE = Ep - E
    b_t = jnp.pad(b8, ((0, 0), (0, padE))).T.reshape(_NT, CH, 8 * _G)
    rows_t = jnp.pad(rows8, ((0, 0), (0, padE))).T.reshape(_NT, CH, 8 * _G)
    dst_t = jnp.pad(edge_index[1], (0, padE),
                    constant_values=N).reshape(_NT, CH, _G)
    b_t = jnp.pad(b_t, ((0, 0), (0, _S), (0, 0)))
    rows_t = jnp.pad(rows_t, ((0, 0), (0, _S), (0, 0)))
    dst_t = jnp.pad(dst_t, ((0, 0), (0, _S), (0, 0)), constant_values=N)

    degp = _make_deg_kernel(n_pad, CH)(dst_t)
    d0 = degp[0, :N]
    d1 = degp[1, :N]

    layers = [(W1, R1, b1), (W2, R2, b2), (W3, R3, b3),
              (W4, R4, b4), (W5, R5, b5), (W6, R6, b6)]
    h = x
    for li, (W, R, b) in enumerate(layers):
        ci, co = W.shape[1], W.shape[2]
        wf = W.transpose(1, 0, 2).reshape(ci, _KD * co)
        if ci == 1:
            xw = _xw_bcast(h, wf)
        else:
            xw = _xw_matmul(h, wf)
        table = xw.reshape(N * _KD, co)
        msgp = _make_msg_kernel(n_pad, CH, co)(table, rows_t, b_t, dst_t)
        h = _post(msgp[0, :N], msgp[1, :N], d0, d1, h, R, b.reshape(1, co))

    return _head(h, fc1_w, fc1_b.reshape(1, -1), fc2_w, fc2_b.reshape(1, -1))
